# baseline (speedup 1.0000x reference)
import functools

import jax
import jax.numpy as jnp
from jax import lax
from jax.experimental import pallas as pl
from jax.experimental.pallas import tpu as pltpu
from jax.experimental.pallas import tpu_sc as plsc

B = 128 * 16 * 16 * 16
D = 32
NC = 2
NS = 16
NW = NC * NS
BPW = B // NW
ROW = 128
NB = 8
NSTEP = BPW // (NB * ROW)

_mesh = plsc.VectorSubcoreMesh(core_axis_name="c", subcore_axis_name="s")


@functools.partial(
    pl.kernel,
    mesh=_mesh,
    compiler_params=pltpu.CompilerParams(use_tc_tiling_on_sc=False),
    out_type=jax.ShapeDtypeStruct((B, D), jnp.float32),
    scratch_types=[
        pltpu.VMEM((BPW,), jnp.int32),
        pltpu.VMEM((2, NB * ROW, D), jnp.float32),
        pltpu.SemaphoreType.DMA,
        pltpu.SemaphoreType.DMA,
    ],
)
def _gather(idx_hbm, table_hbm, out_hbm, idx_v, rbuf, gsem, osem):
    wid = lax.axis_index("s") * NC + lax.axis_index("c")
    base = wid * BPW
    pltpu.sync_copy(idx_hbm.at[pl.ds(base, BPW)], idx_v)

    def fire_gathers(s, buf):
        hs = []
        for b in range(NB):
            j = s * NB + b
            hs.append(
                pltpu.async_copy(
                    table_hbm.at[idx_v.at[pl.ds(j * ROW, ROW)]],
                    buf.at[pl.ds(b * ROW, ROW)],
                    gsem,
                )
            )
        return hs

    gh = fire_gathers(0, rbuf.at[0])
    wh = {}
    for s in range(NSTEP):
        cur = rbuf.at[s % 2]
        if s + 1 < NSTEP:
            if s >= 1:
                wh[s - 1].wait()
            nxt_gh = fire_gathers(s + 1, rbuf.at[(s + 1) % 2])
        for h in gh:
            h.wait()
        wh[s] = pltpu.async_copy(
            cur, out_hbm.at[pl.ds(base + s * NB * ROW, NB * ROW)], osem
        )
        if s + 1 < NSTEP:
            gh = nxt_gh
    wh[NSTEP - 2].wait()
    wh[NSTEP - 1].wait()


def kernel(inputs, table):
    idxp = (
        inputs.reshape(B // 256, 2, 8, 4, 4)
        .transpose(0, 1, 3, 2, 4)
        .reshape(-1)
    )
    z = _gather(idxp, table)
    return (
        z.reshape(B // 256, 2, 4, 8, 4, 32)
        .transpose(0, 1, 3, 2, 4, 5)
        .reshape(inputs.shape[0], inputs.shape[1], inputs.shape[2], 16 * D)
    )


# speedup vs baseline: 3.3376x; 3.3376x over previous
import functools

import jax
import jax.numpy as jnp
from jax import lax
from jax.experimental import pallas as pl
from jax.experimental.pallas import tpu as pltpu
from jax.experimental.pallas import tpu_sc as plsc

B = 128 * 16 * 16 * 16
D = 32
NC = 2
NS = 16
NW = NC * NS
ROW = 128
NB = 8
K = 2
BCH = B // K
BPW = BCH // NW
NSTEP = BPW // (NB * ROW)

_mesh = plsc.VectorSubcoreMesh(core_axis_name="c", subcore_axis_name="s")


def _make_chunk(c):
    @functools.partial(
        pl.kernel,
        mesh=_mesh,
        compiler_params=pltpu.CompilerParams(use_tc_tiling_on_sc=False),
        out_type=jax.ShapeDtypeStruct((BCH, D), jnp.float32),
        scratch_types=[
            pltpu.VMEM((BPW,), jnp.int32),
            pltpu.VMEM((2, NB * ROW, D), jnp.float32),
            pltpu.SemaphoreType.DMA,
            pltpu.SemaphoreType.DMA,
        ],
    )
    def _gather(idx_hbm, table_hbm, out_hbm, idx_v, rbuf, gsem, osem):
        wid = lax.axis_index("s") * NC + lax.axis_index("c")
        base = wid * BPW
        pltpu.sync_copy(idx_hbm.at[pl.ds(c * BCH + base, BPW)], idx_v)

        def fire_gathers(s, buf):
            hs = []
            for b in range(NB):
                j = s * NB + b
                hs.append(
                    pltpu.async_copy(
                        table_hbm.at[idx_v.at[pl.ds(j * ROW, ROW)]],
                        buf.at[pl.ds(b * ROW, ROW)],
                        gsem,
                    )
                )
            return hs

        gh = fire_gathers(0, rbuf.at[0])
        wh = {}
        for s in range(NSTEP):
            cur = rbuf.at[s % 2]
            if s + 1 < NSTEP:
                if s >= 1:
                    wh[s - 1].wait()
                nxt_gh = fire_gathers(s + 1, rbuf.at[(s + 1) % 2])
            for h in gh:
                h.wait()
            wh[s] = pltpu.async_copy(
                cur, out_hbm.at[pl.ds(base + s * NB * ROW, NB * ROW)], osem
            )
            if s + 1 < NSTEP:
                gh = nxt_gh
        if NSTEP >= 2:
            wh[NSTEP - 2].wait()
        wh[NSTEP - 1].wait()

    return _gather


_chunks = [_make_chunk(c) for c in range(K)]


def kernel(inputs, table):
    idx = inputs.reshape(-1)
    b, x, y = inputs.shape[0], inputs.shape[1], inputs.shape[2]
    parts = [
        _chunks[c](idx, table).reshape(b // K, x, y, 16 * D) for c in range(K)
    ]
    return jnp.concatenate(parts, axis=0)


# speedup vs baseline: 3.8831x; 1.1634x over previous
import functools

import jax
import jax.numpy as jnp
from jax import lax
from jax.experimental import pallas as pl
from jax.experimental.pallas import tpu as pltpu
from jax.experimental.pallas import tpu_sc as plsc

B = 128 * 16 * 16 * 16
D = 32
NC = 2
NS = 16
NW = NC * NS
BPW = B // NW
ROW = 128
NB = 8
NSTEP = BPW // (NB * ROW)

_mesh = plsc.VectorSubcoreMesh(core_axis_name="c", subcore_axis_name="s")


@functools.partial(
    pl.kernel,
    mesh=_mesh,
    compiler_params=pltpu.CompilerParams(use_tc_tiling_on_sc=False),
    out_type=jax.ShapeDtypeStruct((B, D), jnp.float32),
    scratch_types=[
        pltpu.VMEM((BPW,), jnp.int32),
        pltpu.VMEM((2, NB * ROW, D), jnp.float32),
        pltpu.SemaphoreType.DMA,
        pltpu.SemaphoreType.DMA,
    ],
)
def _gather(idx_hbm, table_hbm, out_hbm, idx_v, rbuf, gsem, osem):
    wid = lax.axis_index("s") * NC + lax.axis_index("c")
    base = wid * BPW
    pltpu.sync_copy(idx_hbm.at[pl.ds(base, BPW)], idx_v)

    def fire_gathers(s, buf):
        hs = []
        for b in range(NB):
            j = s * NB + b
            hs.append(
                pltpu.async_copy(
                    table_hbm.at[idx_v.at[pl.ds(j * ROW, ROW)]],
                    buf.at[pl.ds(b * ROW, ROW)],
                    gsem,
                )
            )
        return hs

    gh = fire_gathers(0, rbuf.at[0])
    wh = {}
    for s in range(NSTEP):
        cur = rbuf.at[s % 2]
        if s + 1 < NSTEP:
            if s >= 1:
                wh[s - 1].wait()
            nxt_gh = fire_gathers(s + 1, rbuf.at[(s + 1) % 2])
        for h in gh:
            h.wait()
        wh[s] = pltpu.async_copy(
            cur, out_hbm.at[pl.ds(base + s * NB * ROW, NB * ROW)], osem
        )
        if s + 1 < NSTEP:
            gh = nxt_gh
    wh[NSTEP - 2].wait()
    wh[NSTEP - 1].wait()


def kernel(inputs, table):
    z = _gather(inputs.reshape(-1), table)
    b, x, y = inputs.shape[0], inputs.shape[1], inputs.shape[2]
    return z.reshape(b, x, y, 16 * D)


# speedup vs baseline: 3.8834x; 1.0001x over previous
import functools

import jax
import jax.numpy as jnp
from jax import lax
from jax.experimental import pallas as pl
from jax.experimental.pallas import tpu as pltpu
from jax.experimental.pallas import tpu_sc as plsc

B = 128 * 16 * 16 * 16
D = 32
NC = 2
NS = 16
NW = NC * NS
BPW = B // NW
ROW = 128
NB = 8
NSTEP = BPW // (NB * ROW)

_mesh = plsc.VectorSubcoreMesh(core_axis_name="c", subcore_axis_name="s")


@functools.partial(
    pl.kernel,
    mesh=_mesh,
    compiler_params=pltpu.CompilerParams(
        use_tc_tiling_on_sc=False, needs_layout_passes=False
    ),
    out_type=jax.ShapeDtypeStruct((B, D), jnp.float32),
    scratch_types=[
        pltpu.VMEM((BPW,), jnp.int32),
        pltpu.VMEM((2, NB * ROW, D), jnp.float32),
        pltpu.SemaphoreType.DMA,
        pltpu.SemaphoreType.DMA,
    ],
)
def _gather(idx_hbm, table_hbm, out_hbm, idx_v, rbuf, gsem, osem):
    wid = lax.axis_index("s") * NC + lax.axis_index("c")
    base = wid * BPW
    pltpu.sync_copy(idx_hbm.at[pl.ds(base, BPW)], idx_v)

    def fire_gathers(s, buf):
        hs = []
        for b in range(NB):
            j = s * NB + b
            hs.append(
                pltpu.async_copy(
                    table_hbm.at[idx_v.at[pl.ds(j * ROW, ROW)]],
                    buf.at[pl.ds(b * ROW, ROW)],
                    gsem,
                )
            )
        return hs

    gh = fire_gathers(0, rbuf.at[0])
    wh = {}
    for s in range(NSTEP):
        cur = rbuf.at[s % 2]
        if s + 1 < NSTEP:
            if s >= 1:
                wh[s - 1].wait()
            nxt_gh = fire_gathers(s + 1, rbuf.at[(s + 1) % 2])
        for h in gh:
            h.wait()
        wh[s] = pltpu.async_copy(
            cur, out_hbm.at[pl.ds(base + s * NB * ROW, NB * ROW)], osem
        )
        if s + 1 < NSTEP:
            gh = nxt_gh
    wh[NSTEP - 2].wait()
    wh[NSTEP - 1].wait()


def kernel(inputs, table):
    z = _gather(inputs.reshape(-1), table)
    b, x, y = inputs.shape[0], inputs.shape[1], inputs.shape[2]
    return z.reshape(b, x, y, 16 * D)


# speedup vs baseline: 3.8867x; 1.0009x over previous
import functools

import jax
import jax.numpy as jnp
from jax import lax
from jax.experimental import pallas as pl
from jax.experimental.pallas import tpu as pltpu
from jax.experimental.pallas import tpu_sc as plsc

B = 128 * 16 * 16 * 16
D = 32
NC = 2
NS = 16
NW = NC * NS
BPW = B // NW
ROW = 128
NB = 8
NSTEP = BPW // (NB * ROW)

_mesh = plsc.VectorSubcoreMesh(core_axis_name="c", subcore_axis_name="s")


@functools.partial(
    pl.kernel,
    mesh=_mesh,
    compiler_params=pltpu.CompilerParams(use_tc_tiling_on_sc=False),
    out_type=jax.ShapeDtypeStruct((B, D), jnp.float32),
    scratch_types=[
        pltpu.VMEM((BPW,), jnp.int32),
        pltpu.VMEM((2, NB * ROW, D), jnp.float32),
        pltpu.SemaphoreType.DMA,
        pltpu.SemaphoreType.DMA,
    ],
)
def _gather(idx_hbm, table_hbm, out_hbm, idx_v, rbuf, gsem, osem):
    wid = lax.axis_index("s") * NC + lax.axis_index("c")
    base = wid * BPW
    pltpu.sync_copy(idx_hbm.at[pl.ds(base, BPW)], idx_v)

    def fire_gathers(s, buf):
        hs = []
        for b in range(NB):
            j = s * NB + b
            hs.append(
                pltpu.async_copy(
                    table_hbm.at[idx_v.at[pl.ds(j * ROW, ROW)]],
                    buf.at[pl.ds(b * ROW, ROW)],
                    gsem,
                )
            )
        return hs

    gh = fire_gathers(0, rbuf.at[0])
    wh = {}
    for s in range(NSTEP):
        cur = rbuf.at[s % 2]
        if s + 1 < NSTEP:
            if s >= 1:
                wh[s - 1].wait()
            nxt_gh = fire_gathers(s + 1, rbuf.at[(s + 1) % 2])
        for h in gh:
            h.wait()
        wh[s] = pltpu.async_copy(
            cur, out_hbm.at[pl.ds(base + s * NB * ROW, NB * ROW)], osem
        )
        if s + 1 < NSTEP:
            gh = nxt_gh
    wh[NSTEP - 2].wait()
    wh[NSTEP - 1].wait()


def kernel(inputs, table):
    z = _gather(inputs.reshape(-1), table)
    b, x, y = inputs.shape[0], inputs.shape[1], inputs.shape[2]
    return z.reshape(b, x, y, 16 * D)


# speedup vs baseline: 3.8920x; 1.0014x over previous
import functools

import jax
import jax.numpy as jnp
from jax import lax
from jax.experimental import pallas as pl
from jax.experimental.pallas import tpu as pltpu
from jax.experimental.pallas import tpu_sc as plsc

B = 128 * 16 * 16 * 16
D = 32
NC = 2
NS = 16
NW = NC * NS
BPW = B // NW
ROW = 64
NB = 16
NSTEP = BPW // (NB * ROW)

_mesh = plsc.VectorSubcoreMesh(core_axis_name="c", subcore_axis_name="s")


@functools.partial(
    pl.kernel,
    mesh=_mesh,
    compiler_params=pltpu.CompilerParams(use_tc_tiling_on_sc=False),
    out_type=jax.ShapeDtypeStruct((B, D), jnp.float32),
    scratch_types=[
        pltpu.VMEM((BPW,), jnp.int32),
        pltpu.VMEM((2, NB * ROW, D), jnp.float32),
        pltpu.SemaphoreType.DMA,
        pltpu.SemaphoreType.DMA,
    ],
)
def _gather(idx_hbm, table_hbm, out_hbm, idx_v, rbuf, gsem, osem):
    wid = lax.axis_index("s") * NC + lax.axis_index("c")
    base = wid * BPW
    pltpu.sync_copy(idx_hbm.at[pl.ds(base, BPW)], idx_v)

    def fire_gathers(s, buf):
        hs = []
        for b in range(NB):
            j = s * NB + b
            hs.append(
                pltpu.async_copy(
                    table_hbm.at[idx_v.at[pl.ds(j * ROW, ROW)]],
                    buf.at[pl.ds(b * ROW, ROW)],
                    gsem,
                )
            )
        return hs

    gh = fire_gathers(0, rbuf.at[0])
    wh = {}
    for s in range(NSTEP):
        cur = rbuf.at[s % 2]
        if s + 1 < NSTEP:
            if s >= 1:
                wh[s - 1].wait()
            nxt_gh = fire_gathers(s + 1, rbuf.at[(s + 1) % 2])
        for h in gh:
            h.wait()
        wh[s] = pltpu.async_copy(
            cur, out_hbm.at[pl.ds(base + s * NB * ROW, NB * ROW)], osem
        )
        if s + 1 < NSTEP:
            gh = nxt_gh
    wh[NSTEP - 2].wait()
    wh[NSTEP - 1].wait()


def kernel(inputs, table):
    z = _gather(inputs.reshape(-1), table)
    b, x, y = inputs.shape[0], inputs.shape[1], inputs.shape[2]
    return z.reshape(b, x, y, 16 * D)
